# baseline (speedup 1.0000x reference)
import dataclasses
import functools

import jax
import jax.numpy as jnp
from jax import lax
from jax.experimental import pallas as pl
from jax.experimental.pallas import tpu as pltpu
from jax.experimental.pallas import tpu_sc as plsc

N_NODES = 10000
N_EDGES = 320000
D = 128

NC = 2
NS = 16
NW = NC * NS
LANES = 16

N_PAD = 10240
EDGES_MAIN = 9984
EDGES_WIN = 10496
UNROLL = 8
GROUPS = EDGES_MAIN // LANES
OUTER = GROUPS // UNROLL
TAIL_GROUPS = (EDGES_WIN - EDGES_MAIN) // LANES

_mesh = plsc.VectorSubcoreMesh(core_axis_name="c", subcore_axis_name="s")

_cp = pltpu.CompilerParams()
if "needs_layout_passes" in pltpu.CompilerParams.__dataclass_fields__:
    _cp = dataclasses.replace(_cp, needs_layout_passes=False)


@functools.partial(
    pl.kernel,
    out_type=jax.ShapeDtypeStruct((NW, N_PAD), jnp.float32),
    mesh=_mesh,
    compiler_params=_cp,
    scratch_types=[
        pltpu.VMEM((2, EDGES_WIN), jnp.int32),
        pltpu.VMEM((N_PAD,), jnp.float32),
    ],
)
def _sc_degree(ei_hbm, degp_hbm, edge_b, hist):
    cid = lax.axis_index("c")
    sid = lax.axis_index("s")
    wid = cid * NS + sid
    base = wid * EDGES_MAIN

    @plsc.parallel_loop(0, N_PAD // LANES, unroll=4)
    def _(m):
        hist[pl.ds(m * LANES, LANES)] = jnp.zeros((LANES,), jnp.float32)

    pltpu.sync_copy(ei_hbm.at[:, pl.ds(base, EDGES_WIN)], edge_b)

    ones = jnp.ones((LANES,), jnp.float32)

    def body(off):
        s16 = edge_b[0, pl.ds(off, LANES)]
        d16 = edge_b[1, pl.ds(off, LANES)]
        plsc.addupdate_scatter(hist, [d16], ones, mask=s16 != d16)

    @plsc.parallel_loop(0, GROUPS, unroll=UNROLL)
    def _(g):
        body(g * LANES)

    @pl.when(wid == NW - 1)
    def _():
        @pl.loop(0, TAIL_GROUPS)
        def _(g):
            body(EDGES_MAIN + g * LANES)

    pltpu.sync_copy(hist, degp_hbm.at[wid])


@functools.partial(
    pl.kernel,
    out_type=jax.ShapeDtypeStruct((NW, N_PAD), jnp.float32),
    mesh=_mesh,
    compiler_params=_cp,
    scratch_types=[
        pltpu.VMEM((2, EDGES_WIN), jnp.int32),
        pltpu.VMEM((N_PAD,), jnp.float32),
        pltpu.VMEM((N_PAD,), jnp.float32),
    ],
)
def _sc_tsum(ei_hbm, dinv_hbm, tp_hbm, edge_b, dinv_b, hist):
    cid = lax.axis_index("c")
    sid = lax.axis_index("s")
    wid = cid * NS + sid
    base = wid * EDGES_MAIN

    @plsc.parallel_loop(0, N_PAD // LANES, unroll=4)
    def _(m):
        hist[pl.ds(m * LANES, LANES)] = jnp.zeros((LANES,), jnp.float32)

    pltpu.sync_copy(ei_hbm.at[:, pl.ds(base, EDGES_WIN)], edge_b)
    pltpu.sync_copy(dinv_hbm, dinv_b)

    def body(off):
        s16 = edge_b[0, pl.ds(off, LANES)]
        d16 = edge_b[1, pl.ds(off, LANES)]
        gv = plsc.load_gather(dinv_b, [d16])
        plsc.addupdate_scatter(hist, [s16], gv, mask=s16 != d16)

    @plsc.parallel_loop(0, GROUPS, unroll=UNROLL)
    def _(g):
        body(g * LANES)

    @pl.when(wid == NW - 1)
    def _():
        @pl.loop(0, TAIL_GROUPS)
        def _(g):
            body(EDGES_MAIN + g * LANES)

    pltpu.sync_copy(hist, tp_hbm.at[wid])


def _mm0_body(g_ref, w_ref, o_ref):
    o_ref[...] = lax.dot_general(g_ref[...], w_ref[...],
                                 (((1,), (1,)), ((), ())),
                                 preferred_element_type=jnp.float32)


def _dinv_body(degp_ref, o_ref):
    deg = jnp.sum(degp_ref[...], axis=0) + 1.0
    o_ref[0, :] = lax.rsqrt(deg)


NB = 10
BLK = 1024


def _dense_body(x1_ref, dinv_ref, tp_ref, w1_ref, b0_ref, b1_ref, o_ref,
                acc_s, acc_q, mean_sc, rstd_sc):
    k = pl.program_id(0)
    b = pl.program_id(1)

    dv = dinv_ref[...]
    t = jnp.sum(tp_ref[...], axis=0, keepdims=True)
    s_rb = dv * (t + dv)
    x1 = x1_ref[...]
    inv_n = 1.0 / N_NODES

    @pl.when(k == 0)
    def _():
        rows_ok = (b * BLK + lax.broadcasted_iota(jnp.int32, (BLK, 1), 0)
                   ) < N_NODES
        x1m = jnp.where(rows_ok, x1, 0.0)
        ps = lax.dot_general(s_rb, x1m, (((1,), (0,)), ((), ())),
                             preferred_element_type=jnp.float32)
        pq = lax.dot_general(s_rb * s_rb, x1m * x1m, (((1,), (0,)), ((), ())),
                             preferred_element_type=jnp.float32)

        @pl.when(b == 0)
        def _():
            acc_s[...] = jnp.zeros_like(acc_s)
            acc_q[...] = jnp.zeros_like(acc_q)

        acc_s[...] += ps
        acc_q[...] += pq

    @pl.when(k == 1)
    def _():
        @pl.when(b == 0)
        def _():
            m = acc_s[...] * inv_n
            v = acc_q[...] * inv_n - m * m
            mean_sc[...] = m
            rstd_sc[...] = lax.rsqrt(v + 1e-5)

        s_col = lax.transpose(s_rb, (1, 0))
        h = jnp.maximum((x1 * s_col - mean_sc[...]) * rstd_sc[...], 0.0)
        y = lax.dot_general(h, w1_ref[...],
                            (((1,), (1,)), ((), ())),
                            preferred_element_type=jnp.float32)
        o_ref[...] = y * s_col + b1_ref[...]


def kernel(graph_node, edge_index, W0, b0, W1, b1):
    degp = _sc_degree(edge_index)

    x1 = pl.pallas_call(
        _mm0_body,
        out_shape=jax.ShapeDtypeStruct((N_NODES, D), jnp.float32),
    )(graph_node, W0)

    dinv = pl.pallas_call(
        _dinv_body,
        out_shape=jax.ShapeDtypeStruct((1, N_PAD), jnp.float32),
    )(degp)

    tp = _sc_tsum(edge_index, dinv.reshape(N_PAD))

    out = pl.pallas_call(
        _dense_body,
        grid=(2, NB),
        in_specs=[
            pl.BlockSpec((BLK, D), lambda k, b: (b, 0)),
            pl.BlockSpec((1, BLK), lambda k, b: (0, b)),
            pl.BlockSpec((NW, BLK), lambda k, b: (0, b)),
            pl.BlockSpec((D, D), lambda k, b: (0, 0)),
            pl.BlockSpec((1, D), lambda k, b: (0, 0)),
            pl.BlockSpec((1, D), lambda k, b: (0, 0)),
        ],
        out_specs=pl.BlockSpec((BLK, D), lambda k, b: (b, 0)),
        scratch_shapes=[
            pltpu.VMEM((1, D), jnp.float32),
            pltpu.VMEM((1, D), jnp.float32),
            pltpu.VMEM((1, D), jnp.float32),
            pltpu.VMEM((1, D), jnp.float32),
        ],
        out_shape=jax.ShapeDtypeStruct((N_NODES, D), jnp.float32),
    )(x1, dinv, tp, W1, b0[None, :], b1[None, :])
    return out


# speedup vs baseline: 1.2662x; 1.2662x over previous
import dataclasses
import functools

import jax
import jax.numpy as jnp
from jax import lax
from jax.experimental import pallas as pl
from jax.experimental.pallas import tpu as pltpu
from jax.experimental.pallas import tpu_sc as plsc

N_NODES = 10000
N_EDGES = 320000
D = 128

NC = 2
NS = 16
NW = NC * NS
LANES = 16

N_PAD = 10240
EDGES_MAIN = 9984
EDGES_WIN = 10496
UNROLL = 8
GROUPS = EDGES_MAIN // LANES
OUTER = GROUPS // UNROLL
TAIL_GROUPS = (EDGES_WIN - EDGES_MAIN) // LANES

_mesh = plsc.VectorSubcoreMesh(core_axis_name="c", subcore_axis_name="s")

_cp = pltpu.CompilerParams()
if "needs_layout_passes" in pltpu.CompilerParams.__dataclass_fields__:
    _cp = dataclasses.replace(_cp, needs_layout_passes=False)


@functools.partial(
    pl.kernel,
    out_type=jax.ShapeDtypeStruct((NW, N_PAD), jnp.float32),
    mesh=_mesh,
    compiler_params=_cp,
    scratch_types=[
        pltpu.VMEM((2, EDGES_WIN), jnp.int32),
        pltpu.VMEM((N_PAD,), jnp.float32),
    ],
)
def _sc_degree(ei_hbm, degp_hbm, edge_b, hist):
    cid = lax.axis_index("c")
    sid = lax.axis_index("s")
    wid = cid * NS + sid
    base = wid * EDGES_MAIN

    @plsc.parallel_loop(0, N_PAD // LANES, unroll=4)
    def _(m):
        hist[pl.ds(m * LANES, LANES)] = jnp.zeros((LANES,), jnp.float32)

    pltpu.sync_copy(ei_hbm.at[:, pl.ds(base, EDGES_WIN)], edge_b)

    ones = jnp.ones((LANES,), jnp.float32)

    def body(off):
        s16 = edge_b[0, pl.ds(off, LANES)]
        d16 = edge_b[1, pl.ds(off, LANES)]
        plsc.addupdate_scatter(hist, [d16], ones, mask=s16 != d16)

    @plsc.parallel_loop(0, GROUPS, unroll=UNROLL)
    def _(g):
        body(g * LANES)

    @pl.when(wid == NW - 1)
    def _():
        @pl.loop(0, TAIL_GROUPS)
        def _(g):
            body(EDGES_MAIN + g * LANES)

    pltpu.sync_copy(hist, degp_hbm.at[wid])


@functools.partial(
    pl.kernel,
    out_type=jax.ShapeDtypeStruct((NW, N_PAD), jnp.float32),
    mesh=_mesh,
    compiler_params=_cp,
    scratch_types=[
        pltpu.VMEM((2, EDGES_WIN), jnp.int32),
        pltpu.VMEM((N_PAD,), jnp.float32),
        pltpu.VMEM((N_PAD,), jnp.float32),
    ],
)
def _sc_tsum(ei_hbm, dinv_hbm, tp_hbm, edge_b, dinv_b, hist):
    cid = lax.axis_index("c")
    sid = lax.axis_index("s")
    wid = cid * NS + sid
    base = wid * EDGES_MAIN

    @plsc.parallel_loop(0, N_PAD // LANES, unroll=4)
    def _(m):
        hist[pl.ds(m * LANES, LANES)] = jnp.zeros((LANES,), jnp.float32)

    pltpu.sync_copy(ei_hbm.at[:, pl.ds(base, EDGES_WIN)], edge_b)
    pltpu.sync_copy(dinv_hbm, dinv_b)

    def body(off):
        s16 = edge_b[0, pl.ds(off, LANES)]
        d16 = edge_b[1, pl.ds(off, LANES)]
        gv = plsc.load_gather(dinv_b, [d16])
        plsc.addupdate_scatter(hist, [s16], gv, mask=s16 != d16)

    @plsc.parallel_loop(0, GROUPS, unroll=UNROLL)
    def _(g):
        body(g * LANES)

    @pl.when(wid == NW - 1)
    def _():
        @pl.loop(0, TAIL_GROUPS)
        def _(g):
            body(EDGES_MAIN + g * LANES)

    pltpu.sync_copy(hist, tp_hbm.at[wid])


def _mm0_body(g_ref, w_ref, o_ref):
    o_ref[...] = lax.dot_general(g_ref[...], w_ref[...],
                                 (((1,), (1,)), ((), ())),
                                 preferred_element_type=jnp.float32)


def _dinv_body(degp_ref, o_ref):
    deg = jnp.sum(degp_ref[...], axis=0) + 1.0
    o_ref[0, :] = lax.rsqrt(deg)


def _dense_body(x1_ref, dinv_ref, tp_ref, w1_ref, b0_ref, b1_ref, o_ref):
    dv = dinv_ref[...]
    t = jnp.sum(tp_ref[...], axis=0, keepdims=True)
    s_full = dv * (t + dv)
    s_row = s_full[:, :N_NODES]
    s = lax.transpose(s_row, (1, 0))
    x1 = x1_ref[...]
    inv_n = 1.0 / N_NODES
    mean_xs = lax.dot_general(s_row, x1, (((1,), (0,)), ((), ())),
                              preferred_element_type=jnp.float32) * inv_n
    ex2 = lax.dot_general(s_row * s_row, x1 * x1, (((1,), (0,)), ((), ())),
                          preferred_element_type=jnp.float32) * inv_n
    v = ex2 - mean_xs * mean_xs
    h = jnp.maximum((x1 * s - mean_xs) * lax.rsqrt(v + 1e-5), 0.0)
    y = lax.dot_general(h, w1_ref[...],
                        (((1,), (1,)), ((), ())),
                        preferred_element_type=jnp.float32)
    o_ref[...] = y * s + b1_ref[...]


def kernel(graph_node, edge_index, W0, b0, W1, b1):
    degp = _sc_degree(edge_index)

    x1 = pl.pallas_call(
        _mm0_body,
        out_shape=jax.ShapeDtypeStruct((N_NODES, D), jnp.float32),
    )(graph_node, W0)

    dinv = pl.pallas_call(
        _dinv_body,
        out_shape=jax.ShapeDtypeStruct((1, N_PAD), jnp.float32),
    )(degp)

    tp = _sc_tsum(edge_index, dinv.reshape(N_PAD))

    out = pl.pallas_call(
        _dense_body,
        out_shape=jax.ShapeDtypeStruct((N_NODES, D), jnp.float32),
    )(x1, dinv, tp, W1, b0[None, :], b1[None, :])
    return out
